# baseline (speedup 1.0000x reference)
import functools

import jax
import jax.numpy as jnp
from jax import lax
from jax.experimental import pallas as pl
from jax.experimental.pallas import tpu as pltpu
from jax.experimental.pallas import tpu_sc as plsc

N = 10000
E = 320000
D = 128
H = 16
C = 40

NC = 2
NS = 16
NW = NC * NS
LANES = 16

EW = E // NW
B = 2000
CH = EW // B

NG = 5
GC = CH // NG

NPAD = 10240
ZR = NPAD // NS

_mesh = plsc.VectorSubcoreMesh(
    core_axis_name="c", subcore_axis_name="s", num_cores=NC, num_subcores=NS
)

_sc_params = pltpu.CompilerParams(use_tc_tiling_on_sc=False)


@functools.partial(
    pl.kernel,
    out_type=jax.ShapeDtypeStruct((NC, NPAD, LANES), jnp.float32),
    mesh=_mesh,
    compiler_params=_sc_params,
    scratch_types=[
        pltpu.VMEM((CH, B), jnp.int32),
        pltpu.VMEM((B, LANES), jnp.float32),
        pltpu.VMEM((ZR, LANES), jnp.float32),
        pltpu.VMEM_SHARED((NPAD, LANES), jnp.float32),
        pltpu.SemaphoreType.DMA,
    ],
)
def _deg_sc(dst_hbm, out_hbm, didx_v, ones_v, zer_v, acc_sh, sem):
    c = lax.axis_index("c")
    s = lax.axis_index("s")
    w = c * NS + s

    @pl.loop(0, B)
    def _(i):
        ones_v[i, :] = jnp.full((LANES,), 1.0, jnp.float32)

    @pl.loop(0, ZR)
    def _(i):
        zer_v[i, :] = jnp.zeros((LANES,), jnp.float32)

    pltpu.sync_copy(zer_v, acc_sh.at[pl.ds(s * ZR, ZR)])
    pltpu.sync_copy(dst_hbm.at[w], didx_v)
    plsc.subcore_barrier()

    @pl.loop(0, CH)
    def _(j):
        pltpu.async_copy(ones_v, acc_sh.at[didx_v.at[j]], sem, add=True)

    @pl.loop(0, CH)
    def _(j):
        pltpu.make_async_copy(ones_v, acc_sh.at[didx_v.at[j]], sem).wait()

    plsc.subcore_barrier()
    pltpu.sync_copy(acc_sh.at[pl.ds(s * ZR, ZR)], out_hbm.at[c, pl.ds(s * ZR, ZR)])


@functools.partial(
    pl.kernel,
    out_type=jax.ShapeDtypeStruct((NC, NPAD, LANES), jnp.float32),
    mesh=_mesh,
    compiler_params=_sc_params,
    scratch_types=[
        pltpu.VMEM((CH, B), jnp.int32),
        pltpu.VMEM((CH, B), jnp.int32),
        pltpu.VMEM((2, GC, B, LANES), jnp.float32),
        pltpu.VMEM((ZR, LANES), jnp.float32),
        pltpu.VMEM_SHARED((NPAD, LANES), jnp.float32),
        pltpu.SemaphoreType.DMA,
        pltpu.SemaphoreType.DMA,
    ],
)
def _agg_sc(tab_hbm, src_hbm, dst_hbm, out_hbm, sidx_v, didx_v, rows_v, zer_v,
            acc_sh, gsem, ssem):
    c = lax.axis_index("c")
    s = lax.axis_index("s")
    w = c * NS + s

    @pl.loop(0, ZR)
    def _(i):
        zer_v[i, :] = jnp.zeros((LANES,), jnp.float32)

    pltpu.sync_copy(zer_v, acc_sh.at[pl.ds(s * ZR, ZR)])
    pltpu.sync_copy(src_hbm.at[w], sidx_v)
    pltpu.sync_copy(dst_hbm.at[w], didx_v)
    plsc.subcore_barrier()

    def fire_gathers(g, b):
        @pl.loop(0, GC)
        def _(k):
            pltpu.async_copy(tab_hbm.at[sidx_v.at[g * GC + k]], rows_v.at[b, k],
                             gsem)

    def drain_gathers():
        @pl.loop(0, GC)
        def _(k):
            pltpu.make_async_copy(tab_hbm.at[sidx_v.at[0]], rows_v.at[0, 0],
                                  gsem).wait()

    def fire_scatters(g, b):
        @pl.loop(0, GC)
        def _(k):
            pltpu.async_copy(rows_v.at[b, k], acc_sh.at[didx_v.at[g * GC + k]],
                             ssem, add=True)

    def drain_scatters():
        @pl.loop(0, GC)
        def _(k):
            pltpu.make_async_copy(rows_v.at[0, 0], acc_sh.at[didx_v.at[0]],
                                  ssem).wait()

    fire_gathers(0, 0)
    for g in range(NG):
        b = g % 2
        drain_gathers()
        if g >= 1:
            drain_scatters()
        if g + 1 < NG:
            fire_gathers(g + 1, 1 - b)
        fire_scatters(g, b)
    drain_scatters()

    plsc.subcore_barrier()
    pltpu.sync_copy(acc_sh.at[pl.ds(s * ZR, ZR)], out_hbm.at[c, pl.ds(s * ZR, ZR)])


_RB = 1000
_GRID = N // _RB


def _mm1_body(x_ref, w_ref, h_ref):
    h_ref[...] = jnp.dot(x_ref[...], w_ref[...],
                         preferred_element_type=jnp.float32)


def _scale_body(h_ref, degp_ref, h1p_ref, dis_ref):
    deg = degp_ref[0] + degp_ref[1] + 1.0
    dis = lax.rsqrt(deg)
    dis_ref[...] = dis
    h1p_ref[...] = h_ref[...] * dis


def _relu_body(aggp_ref, h1p_ref, dis_ref, b1_ref, zp_ref):
    dis = dis_ref[...]
    pre = dis * (aggp_ref[0] + aggp_ref[1] + h1p_ref[...]) + b1_ref[...]
    zp_ref[...] = dis * jnp.maximum(pre, 0.0)


def _final_body(aggp_ref, zp_ref, dis_ref, w2_ref, b2_ref, out_ref):
    t = dis_ref[...] * (aggp_ref[0] + aggp_ref[1] + zp_ref[...])
    out_ref[...] = jnp.dot(t, w2_ref[...],
                           preferred_element_type=jnp.float32) + b2_ref[...]


def _row_spec(width):
    return pl.BlockSpec((_RB, width), lambda i: (i, 0))


def _part_spec():
    return pl.BlockSpec((NC, _RB, LANES), lambda i: (0, i, 0))


def _full_spec(shape):
    return pl.BlockSpec(shape, lambda i: tuple(0 for _ in shape))


@jax.jit
def kernel(x, edge_index, W1, b1, W2, b2):
    src_rs = edge_index[0].reshape(NW, CH, B)
    dst_rs = edge_index[1].reshape(NW, CH, B)

    degp = _deg_sc(dst_rs)

    h1 = pl.pallas_call(
        _mm1_body,
        grid=(_GRID,),
        in_specs=[_row_spec(D), _full_spec((D, H))],
        out_specs=_row_spec(H),
        out_shape=jax.ShapeDtypeStruct((N, H), jnp.float32),
    )(x, W1)

    h1p, dis = pl.pallas_call(
        _scale_body,
        grid=(_GRID,),
        in_specs=[_row_spec(H), _part_spec()],
        out_specs=[_row_spec(H), _row_spec(H)],
        out_shape=[
            jax.ShapeDtypeStruct((N, H), jnp.float32),
            jax.ShapeDtypeStruct((N, H), jnp.float32),
        ],
    )(h1, degp)

    agg1p = _agg_sc(h1p, src_rs, dst_rs)

    zp = pl.pallas_call(
        _relu_body,
        grid=(_GRID,),
        in_specs=[_part_spec(), _row_spec(H), _row_spec(H),
                  _full_spec((1, H))],
        out_specs=_row_spec(H),
        out_shape=jax.ShapeDtypeStruct((N, H), jnp.float32),
    )(agg1p, h1p, dis, b1.reshape(1, H))

    agg2p = _agg_sc(zp, src_rs, dst_rs)

    out = pl.pallas_call(
        _final_body,
        grid=(_GRID,),
        in_specs=[_part_spec(), _row_spec(H), _row_spec(H),
                  _full_spec((H, C)), _full_spec((1, C))],
        out_specs=_row_spec(C),
        out_shape=jax.ShapeDtypeStruct((N, C), jnp.float32),
    )(agg2p, zp, dis, W2, b2.reshape(1, C))

    return out


# speedup vs baseline: 1.0349x; 1.0349x over previous
import functools

import jax
import jax.numpy as jnp
from jax import lax
from jax.experimental import pallas as pl
from jax.experimental.pallas import tpu as pltpu
from jax.experimental.pallas import tpu_sc as plsc

N = 10000
E = 320000
D = 128
H = 16
C = 40

NC = 2
NS = 16
NW = NC * NS
LANES = 16

EW = E // NW
B = 2000
CH = EW // B

NG = 5
GC = CH // NG

NPAD = 10240
ZR = NPAD // NS

_mesh = plsc.VectorSubcoreMesh(
    core_axis_name="c", subcore_axis_name="s", num_cores=NC, num_subcores=NS
)

_sc_params = pltpu.CompilerParams(use_tc_tiling_on_sc=False)


@functools.partial(
    pl.kernel,
    out_type=jax.ShapeDtypeStruct((NC, NPAD, LANES), jnp.float32),
    mesh=_mesh,
    compiler_params=_sc_params,
    scratch_types=[
        pltpu.VMEM((CH, B), jnp.int32),
        pltpu.VMEM((B, LANES), jnp.float32),
        pltpu.VMEM((ZR, LANES), jnp.float32),
        pltpu.VMEM_SHARED((NPAD, LANES), jnp.float32),
        pltpu.SemaphoreType.DMA,
    ],
)
def _deg_sc(dst_hbm, out_hbm, didx_v, ones_v, zer_v, acc_sh, sem):
    c = lax.axis_index("c")
    s = lax.axis_index("s")
    w = c * NS + s

    @pl.loop(0, B)
    def _(i):
        ones_v[i, :] = jnp.full((LANES,), 1.0, jnp.float32)

    @pl.loop(0, ZR)
    def _(i):
        zer_v[i, :] = jnp.zeros((LANES,), jnp.float32)

    pltpu.sync_copy(zer_v, acc_sh.at[pl.ds(s * ZR, ZR)])
    pltpu.sync_copy(dst_hbm.at[w], didx_v)
    plsc.subcore_barrier()

    @pl.loop(0, CH)
    def _(j):
        pltpu.async_copy(ones_v, acc_sh.at[didx_v.at[j]], sem, add=True)

    @pl.loop(0, CH)
    def _(j):
        pltpu.make_async_copy(ones_v, acc_sh.at[didx_v.at[j]], sem).wait()

    plsc.subcore_barrier()
    pltpu.sync_copy(acc_sh.at[pl.ds(s * ZR, ZR)], out_hbm.at[c, pl.ds(s * ZR, ZR)])


@functools.partial(
    pl.kernel,
    out_type=jax.ShapeDtypeStruct((NC, NPAD, LANES), jnp.float32),
    mesh=_mesh,
    compiler_params=_sc_params,
    scratch_types=[
        pltpu.VMEM((CH, B), jnp.int32),
        pltpu.VMEM((CH, B), jnp.int32),
        pltpu.VMEM((2, GC, B, LANES), jnp.float32),
        pltpu.VMEM((ZR, LANES), jnp.float32),
        pltpu.VMEM_SHARED((NPAD, LANES), jnp.float32),
        pltpu.VMEM_SHARED((N, LANES), jnp.float32),
        pltpu.SemaphoreType.DMA,
        pltpu.SemaphoreType.DMA,
    ],
)
def _agg_sc(tab_hbm, src_hbm, dst_hbm, out_hbm, sidx_v, didx_v, rows_v, zer_v,
            acc_sh, tab_sh, gsem, ssem):
    c = lax.axis_index("c")
    s = lax.axis_index("s")
    w = c * NS + s

    @pl.loop(0, ZR)
    def _(i):
        zer_v[i, :] = jnp.zeros((LANES,), jnp.float32)

    pltpu.sync_copy(zer_v, acc_sh.at[pl.ds(s * ZR, ZR)])
    pltpu.sync_copy(src_hbm.at[w], sidx_v)
    pltpu.sync_copy(dst_hbm.at[w], didx_v)
    pltpu.sync_copy(tab_hbm.at[pl.ds(s * (N // NS), N // NS)],
                    tab_sh.at[pl.ds(s * (N // NS), N // NS)])
    plsc.subcore_barrier()

    def fire_gathers(g, b):
        @pl.loop(0, GC)
        def _(k):
            pltpu.async_copy(tab_sh.at[sidx_v.at[g * GC + k]], rows_v.at[b, k],
                             gsem)

    def drain_gathers():
        @pl.loop(0, GC)
        def _(k):
            pltpu.make_async_copy(tab_sh.at[sidx_v.at[0]], rows_v.at[0, 0],
                                  gsem).wait()

    def fire_scatters(g, b):
        @pl.loop(0, GC)
        def _(k):
            pltpu.async_copy(rows_v.at[b, k], acc_sh.at[didx_v.at[g * GC + k]],
                             ssem, add=True)

    def drain_scatters():
        @pl.loop(0, GC)
        def _(k):
            pltpu.make_async_copy(rows_v.at[0, 0], acc_sh.at[didx_v.at[0]],
                                  ssem).wait()

    fire_gathers(0, 0)
    for g in range(NG):
        b = g % 2
        drain_gathers()
        if g >= 1:
            drain_scatters()
        if g + 1 < NG:
            fire_gathers(g + 1, 1 - b)
        fire_scatters(g, b)
    drain_scatters()

    plsc.subcore_barrier()
    pltpu.sync_copy(acc_sh.at[pl.ds(s * ZR, ZR)], out_hbm.at[c, pl.ds(s * ZR, ZR)])


_RB = 1000
_GRID = N // _RB


def _mm1_body(x_ref, w_ref, h_ref):
    h_ref[...] = jnp.dot(x_ref[...], w_ref[...],
                         preferred_element_type=jnp.float32)


def _scale_body(h_ref, degp_ref, h1p_ref, dis_ref):
    deg = degp_ref[0] + degp_ref[1] + 1.0
    dis = lax.rsqrt(deg)
    dis_ref[...] = dis
    h1p_ref[...] = h_ref[...] * dis


def _relu_body(aggp_ref, h1p_ref, dis_ref, b1_ref, zp_ref):
    dis = dis_ref[...]
    pre = dis * (aggp_ref[0] + aggp_ref[1] + h1p_ref[...]) + b1_ref[...]
    zp_ref[...] = dis * jnp.maximum(pre, 0.0)


def _final_body(aggp_ref, zp_ref, dis_ref, w2_ref, b2_ref, out_ref):
    t = dis_ref[...] * (aggp_ref[0] + aggp_ref[1] + zp_ref[...])
    out_ref[...] = jnp.dot(t, w2_ref[...],
                           preferred_element_type=jnp.float32) + b2_ref[...]


def _row_spec(width):
    return pl.BlockSpec((_RB, width), lambda i: (i, 0))


def _part_spec():
    return pl.BlockSpec((NC, _RB, LANES), lambda i: (0, i, 0))


def _full_spec(shape):
    return pl.BlockSpec(shape, lambda i: tuple(0 for _ in shape))


@jax.jit
def kernel(x, edge_index, W1, b1, W2, b2):
    src_rs = edge_index[0].reshape(NW, CH, B)
    dst_rs = edge_index[1].reshape(NW, CH, B)

    degp = _deg_sc(dst_rs)

    h1 = pl.pallas_call(
        _mm1_body,
        grid=(_GRID,),
        in_specs=[_row_spec(D), _full_spec((D, H))],
        out_specs=_row_spec(H),
        out_shape=jax.ShapeDtypeStruct((N, H), jnp.float32),
    )(x, W1)

    h1p, dis = pl.pallas_call(
        _scale_body,
        grid=(_GRID,),
        in_specs=[_row_spec(H), _part_spec()],
        out_specs=[_row_spec(H), _row_spec(H)],
        out_shape=[
            jax.ShapeDtypeStruct((N, H), jnp.float32),
            jax.ShapeDtypeStruct((N, H), jnp.float32),
        ],
    )(h1, degp)

    agg1p = _agg_sc(h1p, src_rs, dst_rs)

    zp = pl.pallas_call(
        _relu_body,
        grid=(_GRID,),
        in_specs=[_part_spec(), _row_spec(H), _row_spec(H),
                  _full_spec((1, H))],
        out_specs=_row_spec(H),
        out_shape=jax.ShapeDtypeStruct((N, H), jnp.float32),
    )(agg1p, h1p, dis, b1.reshape(1, H))

    agg2p = _agg_sc(zp, src_rs, dst_rs)

    out = pl.pallas_call(
        _final_body,
        grid=(_GRID,),
        in_specs=[_part_spec(), _row_spec(H), _row_spec(H),
                  _full_spec((H, C)), _full_spec((1, C))],
        out_specs=_row_spec(C),
        out_shape=jax.ShapeDtypeStruct((N, C), jnp.float32),
    )(agg2p, zp, dis, W2, b2.reshape(1, C))

    return out


# speedup vs baseline: 1.2891x; 1.2457x over previous
import functools

import jax
import jax.numpy as jnp
from jax import lax
from jax.experimental import pallas as pl
from jax.experimental.pallas import tpu as pltpu
from jax.experimental.pallas import tpu_sc as plsc

N = 10000
E = 320000
D = 128
H = 16
C = 40

NC = 2
NS = 16
NW = NC * NS
LANES = 16

B = 128
CH = 80
EP = NW * CH * B
DUMMY = N

NG = 8
GC = CH // NG

NPAD = 10240
ZR = NPAD // NS

PR = N * LANES // 128
PRP = NPAD * LANES // 128

_mesh = plsc.VectorSubcoreMesh(
    core_axis_name="c", subcore_axis_name="s", num_cores=NC, num_subcores=NS
)

_sc_params = pltpu.CompilerParams(use_tc_tiling_on_sc=False)


@functools.partial(
    pl.kernel,
    out_type=jax.ShapeDtypeStruct((NC, NPAD, LANES), jnp.float32),
    mesh=_mesh,
    compiler_params=_sc_params,
    scratch_types=[
        pltpu.VMEM((CH, B), jnp.int32),
        pltpu.VMEM((B, LANES), jnp.float32),
        pltpu.VMEM((ZR, LANES), jnp.float32),
        pltpu.VMEM_SHARED((NPAD, LANES), jnp.float32),
        pltpu.SemaphoreType.DMA,
    ],
)
def _deg_sc(dst_hbm, out_hbm, didx_v, ones_v, zer_v, acc_sh, sem):
    c = lax.axis_index("c")
    s = lax.axis_index("s")
    w = c * NS + s

    @pl.loop(0, B)
    def _(i):
        ones_v[i, :] = jnp.full((LANES,), 1.0, jnp.float32)

    @pl.loop(0, ZR)
    def _(i):
        zer_v[i, :] = jnp.zeros((LANES,), jnp.float32)

    pltpu.sync_copy(zer_v, acc_sh.at[pl.ds(s * ZR, ZR)])
    pltpu.sync_copy(dst_hbm.at[pl.ds(w * CH, CH)], didx_v)
    plsc.subcore_barrier()

    @pl.loop(0, CH)
    def _(j):
        pltpu.async_copy(ones_v, acc_sh.at[didx_v.at[j]], sem, add=True)

    @pl.loop(0, CH)
    def _(j):
        pltpu.make_async_copy(ones_v, acc_sh.at[didx_v.at[j]], sem).wait()

    plsc.subcore_barrier()
    pltpu.sync_copy(acc_sh.at[pl.ds(s * ZR, ZR)], out_hbm.at[c, pl.ds(s * ZR, ZR)])


@functools.partial(
    pl.kernel,
    out_type=jax.ShapeDtypeStruct((NC, NPAD, LANES), jnp.float32),
    mesh=_mesh,
    compiler_params=_sc_params,
    scratch_types=[
        pltpu.VMEM((CH, B), jnp.int32),
        pltpu.VMEM((CH, B), jnp.int32),
        pltpu.VMEM((2, GC, B, LANES), jnp.float32),
        pltpu.VMEM((ZR, LANES), jnp.float32),
        pltpu.VMEM_SHARED((NPAD, LANES), jnp.float32),
        pltpu.VMEM_SHARED((NPAD, LANES), jnp.float32),
        pltpu.SemaphoreType.DMA,
        pltpu.SemaphoreType.DMA,
    ],
)
def _agg_sc(tab_hbm, src_hbm, dst_hbm, out_hbm, sidx_v, didx_v, rows_v, zer_v,
            acc_sh, tab_sh, gsem, ssem):
    c = lax.axis_index("c")
    s = lax.axis_index("s")
    w = c * NS + s

    @pl.loop(0, ZR)
    def _(i):
        zer_v[i, :] = jnp.zeros((LANES,), jnp.float32)

    pltpu.sync_copy(zer_v, acc_sh.at[pl.ds(s * ZR, ZR)])
    pltpu.sync_copy(src_hbm.at[pl.ds(w * CH, CH)], sidx_v)
    pltpu.sync_copy(dst_hbm.at[pl.ds(w * CH, CH)], didx_v)
    pltpu.sync_copy(tab_hbm.at[pl.ds(s * ZR, ZR)], tab_sh.at[pl.ds(s * ZR, ZR)])
    plsc.subcore_barrier()

    def fire_gathers(g, b):
        @pl.loop(0, GC)
        def _(k):
            pltpu.async_copy(tab_sh.at[sidx_v.at[g * GC + k]], rows_v.at[b, k],
                             gsem)

    def drain_gathers():
        @pl.loop(0, GC)
        def _(k):
            pltpu.make_async_copy(tab_sh.at[sidx_v.at[0]], rows_v.at[0, 0],
                                  gsem).wait()

    def fire_scatters(g, b):
        @pl.loop(0, GC)
        def _(k):
            pltpu.async_copy(rows_v.at[b, k], acc_sh.at[didx_v.at[g * GC + k]],
                             ssem, add=True)

    def drain_scatters():
        @pl.loop(0, GC)
        def _(k):
            pltpu.make_async_copy(rows_v.at[0, 0], acc_sh.at[didx_v.at[0]],
                                  ssem).wait()

    fire_gathers(0, 0)
    for g in range(NG):
        b = g % 2
        drain_gathers()
        if g >= 1:
            drain_scatters()
        if g + 1 < NG:
            fire_gathers(g + 1, 1 - b)
        fire_scatters(g, b)
    drain_scatters()

    plsc.subcore_barrier()
    pltpu.sync_copy(acc_sh.at[pl.ds(s * ZR, ZR)], out_hbm.at[c, pl.ds(s * ZR, ZR)])


def _mm1_body(x_ref, wbig_ref, degp_ref, h1p_ref, dis_ref):
    deg = degp_ref[0] + degp_ref[1] + 1.0
    dis = lax.rsqrt(deg)
    dis_ref[...] = dis
    h = jnp.dot(x_ref[...], wbig_ref[...], preferred_element_type=jnp.float32)
    h1p_ref[...] = h * dis


def _relu_body(aggp_ref, h1p_ref, dis_ref, b1_ref, zp_ref):
    dis = dis_ref[...]
    pre = dis * (aggp_ref[0] + aggp_ref[1] + h1p_ref[...]) + b1_ref[...]
    zp_ref[...] = dis * jnp.maximum(pre, 0.0)


def _final_body(aggp_ref, zp_ref, dis_ref, w2big_ref, b2_ref, out_ref):
    t = dis_ref[...] * (aggp_ref[0] + aggp_ref[1] + zp_ref[...])
    out_ref[...] = jnp.dot(t, w2big_ref[...],
                           preferred_element_type=jnp.float32) + b2_ref[...]


def _blk(shape):
    return pl.BlockSpec(shape, lambda: tuple(0 for _ in shape))


@jax.jit
def kernel(x, edge_index, W1, b1, W2, b2):
    pad = jnp.full((EP - E,), DUMMY, jnp.int32)
    srcp = jnp.concatenate([edge_index[0], pad]).reshape(NW * CH, B)
    dstp = jnp.concatenate([edge_index[1], pad]).reshape(NW * CH, B)

    eye8 = jnp.eye(8, dtype=jnp.float32)
    wbig = jnp.kron(eye8, W1)
    w2big = jnp.kron(eye8, W2)
    b1t = jnp.tile(b1, 8).reshape(1, 128)
    b2t = jnp.tile(b2, 8).reshape(1, 320)

    degp = _deg_sc(dstp)

    x_pad = jnp.pad(x.reshape(PR, 1024), ((0, PRP - PR), (0, 0)))
    h1pp, disp = pl.pallas_call(
        _mm1_body,
        in_specs=[_blk((PRP, 1024)), _blk((1024, 128)), _blk((NC, PRP, 128))],
        out_specs=[_blk((PRP, 128)), _blk((PRP, 128))],
        out_shape=[
            jax.ShapeDtypeStruct((PRP, 128), jnp.float32),
            jax.ShapeDtypeStruct((PRP, 128), jnp.float32),
        ],
    )(x_pad, wbig, degp.reshape(NC, PRP, 128))

    agg1p = _agg_sc(h1pp.reshape(NPAD, LANES), srcp, dstp)

    zpp = pl.pallas_call(
        _relu_body,
        in_specs=[_blk((NC, PRP, 128)), _blk((PRP, 128)), _blk((PRP, 128)),
                  _blk((1, 128))],
        out_specs=_blk((PRP, 128)),
        out_shape=jax.ShapeDtypeStruct((PRP, 128), jnp.float32),
    )(agg1p.reshape(NC, PRP, 128), h1pp, disp, b1t)

    agg2p = _agg_sc(zpp.reshape(NPAD, LANES), srcp, dstp)

    outg = pl.pallas_call(
        _final_body,
        in_specs=[
            _blk((NC, PRP, 128)), _blk((PRP, 128)), _blk((PRP, 128)),
            _blk((128, 320)), _blk((1, 320)),
        ],
        out_specs=_blk((PRP, 320)),
        out_shape=jax.ShapeDtypeStruct((PRP, 320), jnp.float32),
    )(agg2p.reshape(NC, PRP, 128), zpp, disp, w2big, b2t)

    return outg.reshape(NPAD, C)[:N]
